# Initial kernel scaffold; baseline (speedup 1.0000x reference)
import jax
import jax.numpy as jnp
from jax.experimental import pallas as pl


def kernel(hidden_states, image_mask, audio_mask, Wg, expert_bias, W_gate, W_up, W_down, Ws_gate, Ws_up, Ws_down):
    raise NotImplementedError("write your pallas kernel here")


# speedup vs baseline: 1.0681x; 1.0681x over previous
import jax
import jax.numpy as jnp
from jax.experimental import pallas as pl

E = 8
NGROUP = 4
GSIZE = E // NGROUP
H = 768
DFF = 384
SCALE = 2.5
N = 2048
BLK = 256
NBLK = N // BLK


def _ranks(vals):
    n = len(vals)
    out = []
    for j in range(n):
        rj = jnp.zeros_like(vals[j])
        for k in range(n):
            if k == j:
                continue
            beats = vals[k] > vals[j]
            if k < j:
                beats = jnp.logical_or(beats, vals[k] == vals[j])
            rj = rj + jnp.where(beats, 1.0, 0.0).astype(jnp.float32)
        out.append(rj)
    return out


def _router_kernel(x_ref, wg_ref, bias_ref, xb_ref, comb_ref):
    x = x_ref[...]
    xb_ref[...] = x.astype(jnp.bfloat16)
    wg = wg_ref[...]
    logits = jax.lax.dot_general(
        x, wg, (((1,), (1,)), ((), ())),
        precision=jax.lax.Precision.DEFAULT,
        preferred_element_type=jnp.float32)
    scores = jax.nn.sigmoid(logits)
    s = scores + bias_ref[...]
    cols = [s[:, e:e + 1] for e in range(E)]
    sc = [scores[:, e:e + 1] for e in range(E)]
    g = [cols[2 * j] + cols[2 * j + 1] for j in range(NGROUP)]
    grank = _ranks(g)
    sel = [grank[j] < 1.5 for j in range(NGROUP)]
    neginf = jnp.float32(-jnp.inf)
    masked = [jnp.where(sel[e // GSIZE], cols[e], neginf) for e in range(E)]
    erank = _ranks(masked)
    top1 = [erank[e] < 0.5 for e in range(E)]
    top2 = [jnp.logical_and(erank[e] > 0.5, erank[e] < 1.5) for e in range(E)]
    zero = jnp.zeros_like(sc[0])
    w1raw = zero
    w2raw = zero
    for e in range(E):
        w1raw = w1raw + jnp.where(top1[e], sc[e], zero)
        w2raw = w2raw + jnp.where(top2[e], sc[e], zero)
    inv = SCALE / (w1raw + w2raw + 1e-20)
    comb = [
        (jnp.where(top1[e], w1raw, zero) + jnp.where(top2[e], w2raw, zero))
        * inv
        for e in range(E)
    ]
    comb_ref[...] = jnp.concatenate(comb, axis=1)


def _moe_kernel(xb_ref, comb_ref, wg_ref, wu_ref, wd_ref,
                wsg_ref, wsu_ref, wsd_ref, out_ref):
    xb = xb_ref[...]
    comb = comb_ref[...]
    acc = jnp.zeros((BLK, H), jnp.float32)
    for e in range(E):
        gm = jnp.dot(xb, wg_ref[e], preferred_element_type=jnp.float32)
        um = jnp.dot(xb, wu_ref[e], preferred_element_type=jnp.float32)
        act = (gm * jax.nn.sigmoid(gm) * um).astype(jnp.bfloat16)
        dm = jnp.dot(act, wd_ref[e], preferred_element_type=jnp.float32)
        acc = acc + comb[:, e:e + 1] * dm
    sg = jnp.dot(xb, wsg_ref[...], preferred_element_type=jnp.float32)
    su = jnp.dot(xb, wsu_ref[...], preferred_element_type=jnp.float32)
    sact = (sg * jax.nn.sigmoid(sg) * su).astype(jnp.bfloat16)
    acc = acc + jnp.dot(sact, wsd_ref[...], preferred_element_type=jnp.float32)
    out_ref[...] = acc


def kernel(hidden_states, image_mask, audio_mask, Wg, expert_bias,
           W_gate, W_up, W_down, Ws_gate, Ws_up, Ws_down):
    orig_shape = hidden_states.shape
    x = hidden_states.reshape(-1, H)
    bias2 = expert_bias.reshape(1, E)

    xb, comb = pl.pallas_call(
        _router_kernel,
        grid=(NBLK,),
        in_specs=[
            pl.BlockSpec((BLK, H), lambda i: (i, 0)),
            pl.BlockSpec((E, H), lambda i: (0, 0)),
            pl.BlockSpec((1, E), lambda i: (0, 0)),
        ],
        out_specs=[
            pl.BlockSpec((BLK, H), lambda i: (i, 0)),
            pl.BlockSpec((BLK, E), lambda i: (i, 0)),
        ],
        out_shape=[
            jax.ShapeDtypeStruct((N, H), jnp.bfloat16),
            jax.ShapeDtypeStruct((N, E), jnp.float32),
        ],
    )(x, Wg, bias2)

    wgT = W_gate.swapaxes(1, 2).astype(jnp.bfloat16)
    wuT = W_up.swapaxes(1, 2).astype(jnp.bfloat16)
    wdT = W_down.swapaxes(1, 2).astype(jnp.bfloat16)
    wsgT = Ws_gate.T.astype(jnp.bfloat16)
    wsuT = Ws_up.T.astype(jnp.bfloat16)
    wsdT = Ws_down.T.astype(jnp.bfloat16)

    out = pl.pallas_call(
        _moe_kernel,
        grid=(NBLK,),
        in_specs=[
            pl.BlockSpec((BLK, H), lambda i: (i, 0)),
            pl.BlockSpec((BLK, E), lambda i: (i, 0)),
            pl.BlockSpec((E, H, DFF), lambda i: (0, 0, 0)),
            pl.BlockSpec((E, H, DFF), lambda i: (0, 0, 0)),
            pl.BlockSpec((E, DFF, H), lambda i: (0, 0, 0)),
            pl.BlockSpec((H, DFF), lambda i: (0, 0)),
            pl.BlockSpec((H, DFF), lambda i: (0, 0)),
            pl.BlockSpec((DFF, H), lambda i: (0, 0)),
        ],
        out_specs=pl.BlockSpec((BLK, H), lambda i: (i, 0)),
        out_shape=jax.ShapeDtypeStruct((N, H), jnp.float32),
    )(xb, comb, wgT, wuT, wdT, wsgT, wsuT, wsdT)

    return out.reshape(orig_shape)


# speedup vs baseline: 1.3060x; 1.2227x over previous
import jax
import jax.numpy as jnp
from jax.experimental import pallas as pl

E = 8
NGROUP = 4
GSIZE = E // NGROUP
H = 768
DFF = 384
SCALE = 2.5
N = 2048
BLK = 256
NBLK = N // BLK

_RHS_T = (((1,), (1,)), ((), ()))


def _ranks(vals):
    n = len(vals)
    out = []
    for j in range(n):
        rj = jnp.zeros_like(vals[j])
        for k in range(n):
            if k == j:
                continue
            beats = vals[k] > vals[j]
            if k < j:
                beats = jnp.logical_or(beats, vals[k] == vals[j])
            rj = rj + jnp.where(beats, 1.0, 0.0).astype(jnp.float32)
        out.append(rj)
    return out


def _routing_weights(x, wg, bias):
    logits = jax.lax.dot_general(
        x, wg, _RHS_T,
        precision=jax.lax.Precision.DEFAULT,
        preferred_element_type=jnp.float32)
    scores = jax.nn.sigmoid(logits)
    s = scores + bias
    cols = [s[:, e:e + 1] for e in range(E)]
    sc = [scores[:, e:e + 1] for e in range(E)]
    g = [cols[2 * j] + cols[2 * j + 1] for j in range(NGROUP)]
    grank = _ranks(g)
    sel = [grank[j] < 1.5 for j in range(NGROUP)]
    neginf = jnp.float32(-jnp.inf)
    masked = [jnp.where(sel[e // GSIZE], cols[e], neginf) for e in range(E)]
    erank = _ranks(masked)
    top1 = [erank[e] < 0.5 for e in range(E)]
    top2 = [jnp.logical_and(erank[e] > 0.5, erank[e] < 1.5) for e in range(E)]
    zero = jnp.zeros_like(sc[0])
    w1raw = zero
    w2raw = zero
    for e in range(E):
        w1raw = w1raw + jnp.where(top1[e], sc[e], zero)
        w2raw = w2raw + jnp.where(top2[e], sc[e], zero)
    inv = SCALE / (w1raw + w2raw + 1e-20)
    return [
        (jnp.where(top1[e], w1raw, zero) + jnp.where(top2[e], w2raw, zero))
        * inv
        for e in range(E)
    ]


def _fused_kernel(x_ref, wgate_ref, bias_ref, wg_ref, wu_ref, wd_ref,
                  wsg_ref, wsu_ref, wsd_ref, out_ref):
    x = x_ref[...]
    comb = _routing_weights(x, wgate_ref[...], bias_ref[...])
    xb = x.astype(jnp.bfloat16)
    acc = jnp.zeros((BLK, H), jnp.float32)
    for e in range(E):
        gm = jax.lax.dot_general(xb, wg_ref[e], _RHS_T,
                                 preferred_element_type=jnp.float32)
        um = jax.lax.dot_general(xb, wu_ref[e], _RHS_T,
                                 preferred_element_type=jnp.float32)
        act = (gm * jax.nn.sigmoid(gm) * um * comb[e]).astype(jnp.bfloat16)
        acc = acc + jax.lax.dot_general(act, wd_ref[e], _RHS_T,
                                        preferred_element_type=jnp.float32)
    sg = jax.lax.dot_general(xb, wsg_ref[...], _RHS_T,
                             preferred_element_type=jnp.float32)
    su = jax.lax.dot_general(xb, wsu_ref[...], _RHS_T,
                             preferred_element_type=jnp.float32)
    sact = (sg * jax.nn.sigmoid(sg) * su).astype(jnp.bfloat16)
    acc = acc + jax.lax.dot_general(sact, wsd_ref[...], _RHS_T,
                                    preferred_element_type=jnp.float32)
    out_ref[...] = acc


def kernel(hidden_states, image_mask, audio_mask, Wg, expert_bias,
           W_gate, W_up, W_down, Ws_gate, Ws_up, Ws_down):
    orig_shape = hidden_states.shape
    x = hidden_states.reshape(-1, H)
    bias2 = expert_bias.reshape(1, E)

    wgb = W_gate.astype(jnp.bfloat16)
    wub = W_up.astype(jnp.bfloat16)
    wdb = W_down.astype(jnp.bfloat16)
    wsgb = Ws_gate.astype(jnp.bfloat16)
    wsub = Ws_up.astype(jnp.bfloat16)
    wsdb = Ws_down.astype(jnp.bfloat16)

    out = pl.pallas_call(
        _fused_kernel,
        grid=(NBLK,),
        in_specs=[
            pl.BlockSpec((BLK, H), lambda i: (i, 0)),
            pl.BlockSpec((E, H), lambda i: (0, 0)),
            pl.BlockSpec((1, E), lambda i: (0, 0)),
            pl.BlockSpec((E, DFF, H), lambda i: (0, 0, 0)),
            pl.BlockSpec((E, DFF, H), lambda i: (0, 0, 0)),
            pl.BlockSpec((E, H, DFF), lambda i: (0, 0, 0)),
            pl.BlockSpec((DFF, H), lambda i: (0, 0)),
            pl.BlockSpec((DFF, H), lambda i: (0, 0)),
            pl.BlockSpec((H, DFF), lambda i: (0, 0)),
        ],
        out_specs=pl.BlockSpec((BLK, H), lambda i: (i, 0)),
        out_shape=jax.ShapeDtypeStruct((N, H), jnp.float32),
    )(x, Wg, bias2, wgb, wub, wdb, wsgb, wsub, wsdb)

    return out.reshape(orig_shape)


# speedup vs baseline: 1.9291x; 1.4771x over previous
import jax
import jax.numpy as jnp
from jax import lax
from jax.experimental import pallas as pl
from jax.experimental.pallas import tpu as pltpu

E = 8
NGROUP = 4
GSIZE = E // NGROUP
H = 768
DFF = 384
SCALE = 2.5
N = 2048

_RHS_T = (((1,), (1,)), ((), ()))


def _routing_cols(x, wg, bias_ref):
    logits = lax.dot_general(
        x, wg, _RHS_T,
        precision=lax.Precision.DEFAULT,
        preferred_element_type=jnp.float32)
    st = jax.nn.sigmoid(logits.T)
    sc = [st[e:e + 1, :] for e in range(E)]
    s = [sc[e] + bias_ref[0, e] for e in range(E)]
    g = [s[2 * j] + s[2 * j + 1] for j in range(NGROUP)]
    grank = []
    for j in range(NGROUP):
        rj = jnp.zeros_like(g[j])
        for k in range(NGROUP):
            if k == j:
                continue
            beats = g[k] > g[j]
            if k < j:
                beats = jnp.logical_or(beats, g[k] == g[j])
            rj = rj + jnp.where(beats, 1.0, 0.0).astype(jnp.float32)
        grank.append(rj)
    sel = [grank[j] < 1.5 for j in range(NGROUP)]
    neginf = jnp.float32(-jnp.inf)
    masked = [jnp.where(sel[e // GSIZE], s[e], neginf) for e in range(E)]
    m1 = masked[0]
    for e in range(1, E):
        m1 = jnp.maximum(m1, masked[e])
    top1 = []
    seen = None
    for e in range(E):
        hit = masked[e] == m1
        top1.append(hit if seen is None else jnp.logical_and(hit, ~seen))
        seen = hit if seen is None else jnp.logical_or(seen, hit)
    rest = [jnp.where(top1[e], neginf, masked[e]) for e in range(E)]
    m2 = rest[0]
    for e in range(1, E):
        m2 = jnp.maximum(m2, rest[e])
    top2 = []
    seen = None
    for e in range(E):
        hit = rest[e] == m2
        top2.append(hit if seen is None else jnp.logical_and(hit, ~seen))
        seen = hit if seen is None else jnp.logical_or(seen, hit)
    zero = jnp.zeros_like(sc[0])
    w1raw = zero
    w2raw = zero
    for e in range(E):
        w1raw = w1raw + jnp.where(top1[e], sc[e], zero)
        w2raw = w2raw + jnp.where(top2[e], sc[e], zero)
    inv = SCALE / (w1raw + w2raw + 1e-20)
    combT = jnp.concatenate(
        [(jnp.where(top1[e], w1raw, zero)
          + jnp.where(top2[e], w2raw, zero)) * inv
         for e in range(E)], axis=0)
    return combT.T


def _fused_kernel(x_ref, wgate_ref, bias_ref, wg_hbm, wu_hbm, wd_hbm,
                  wsg_hbm, wsu_hbm, wsd_hbm, out_ref,
                  g_stg, u_stg, d_stg, xb_ref, comb_ref, sems):
    def copies(e, b):
        if e < E:
            return (pltpu.make_async_copy(wg_hbm.at[e], g_stg.at[b], sems.at[b]),
                    pltpu.make_async_copy(wu_hbm.at[e], u_stg.at[b], sems.at[b]),
                    pltpu.make_async_copy(wd_hbm.at[e], d_stg.at[b], sems.at[b]))
        return (pltpu.make_async_copy(wsg_hbm, g_stg.at[b], sems.at[b]),
                pltpu.make_async_copy(wsu_hbm, u_stg.at[b], sems.at[b]),
                pltpu.make_async_copy(wsd_hbm, d_stg.at[b], sems.at[b]))

    for c in copies(0, 0):
        c.start()

    x = x_ref[...]
    comb_ref[...] = _routing_cols(x, wgate_ref[...], bias_ref)
    xb_ref[...] = x.astype(jnp.bfloat16)
    for e in range(E + 1):
        b = e % 2
        if e < E:
            for c in copies(e + 1, 1 - b):
                c.start()
        for c in copies(e, b):
            c.wait()
        xb = xb_ref[...]
        wgb = g_stg[b].astype(jnp.bfloat16)
        wub = u_stg[b].astype(jnp.bfloat16)
        wdb = d_stg[b].astype(jnp.bfloat16)
        gm = lax.dot_general(xb, wgb, _RHS_T,
                             preferred_element_type=jnp.float32)
        um = lax.dot_general(xb, wub, _RHS_T,
                             preferred_element_type=jnp.float32)
        act = gm * jax.nn.sigmoid(gm) * um
        if e < E:
            act = act * comb_ref[:, e:e + 1]
        actb = act.astype(jnp.bfloat16)
        dm = lax.dot_general(actb, wdb, _RHS_T,
                             preferred_element_type=jnp.float32)
        if e == 0:
            out_ref[...] = dm
        else:
            out_ref[...] = out_ref[...] + dm


def kernel(hidden_states, image_mask, audio_mask, Wg, expert_bias,
           W_gate, W_up, W_down, Ws_gate, Ws_up, Ws_down):
    orig_shape = hidden_states.shape
    x = hidden_states.reshape(-1, H)
    bias2 = expert_bias.reshape(1, E)

    out = pl.pallas_call(
        _fused_kernel,
        in_specs=[
            pl.BlockSpec((N, H), lambda: (0, 0)),
            pl.BlockSpec((E, H), lambda: (0, 0)),
            pl.BlockSpec((1, E), lambda: (0, 0)),
            pl.BlockSpec(memory_space=pl.ANY),
            pl.BlockSpec(memory_space=pl.ANY),
            pl.BlockSpec(memory_space=pl.ANY),
            pl.BlockSpec(memory_space=pl.ANY),
            pl.BlockSpec(memory_space=pl.ANY),
            pl.BlockSpec(memory_space=pl.ANY),
        ],
        out_specs=pl.BlockSpec((N, H), lambda: (0, 0)),
        out_shape=jax.ShapeDtypeStruct((N, H), jnp.float32),
        scratch_shapes=[
            pltpu.VMEM((2, DFF, H), jnp.float32),
            pltpu.VMEM((2, DFF, H), jnp.float32),
            pltpu.VMEM((2, H, DFF), jnp.float32),
            pltpu.VMEM((N, H), jnp.bfloat16),
            pltpu.VMEM((N, E), jnp.float32),
            pltpu.SemaphoreType.DMA((2,)),
        ],
    )(x, Wg, bias2, W_gate, W_up, W_down, Ws_gate, Ws_up, Ws_down)

    return out.reshape(orig_shape)


# speedup vs baseline: 2.0078x; 1.0408x over previous
import jax
import jax.numpy as jnp
from jax import lax
from jax.experimental import pallas as pl
from jax.experimental.pallas import tpu as pltpu

E = 8
NGROUP = 4
GSIZE = E // NGROUP
H = 768
DFF = 384
SCALE = 2.5
N = 2048

_RHS_T = (((1,), (1,)), ((), ()))


def _routing_cols(x, wg, bias_ref):
    logits = lax.dot_general(
        x, wg, _RHS_T,
        precision=lax.Precision.DEFAULT,
        preferred_element_type=jnp.float32)
    st = jax.nn.sigmoid(logits.T)
    sc = [st[e:e + 1, :] for e in range(E)]
    s = [sc[e] + bias_ref[0, e] for e in range(E)]
    g = [s[2 * j] + s[2 * j + 1] for j in range(NGROUP)]
    grank = []
    for j in range(NGROUP):
        rj = jnp.zeros_like(g[j])
        for k in range(NGROUP):
            if k == j:
                continue
            beats = g[k] > g[j]
            if k < j:
                beats = jnp.logical_or(beats, g[k] == g[j])
            rj = rj + jnp.where(beats, 1.0, 0.0).astype(jnp.float32)
        grank.append(rj)
    sel = [grank[j] < 1.5 for j in range(NGROUP)]
    neginf = jnp.float32(-jnp.inf)
    masked = [jnp.where(sel[e // GSIZE], s[e], neginf) for e in range(E)]
    m1 = masked[0]
    for e in range(1, E):
        m1 = jnp.maximum(m1, masked[e])
    top1 = []
    seen = None
    for e in range(E):
        hit = masked[e] == m1
        top1.append(hit if seen is None else jnp.logical_and(hit, ~seen))
        seen = hit if seen is None else jnp.logical_or(seen, hit)
    rest = [jnp.where(top1[e], neginf, masked[e]) for e in range(E)]
    m2 = rest[0]
    for e in range(1, E):
        m2 = jnp.maximum(m2, rest[e])
    top2 = []
    seen = None
    for e in range(E):
        hit = rest[e] == m2
        top2.append(hit if seen is None else jnp.logical_and(hit, ~seen))
        seen = hit if seen is None else jnp.logical_or(seen, hit)
    zero = jnp.zeros_like(sc[0])
    w1raw = zero
    w2raw = zero
    for e in range(E):
        w1raw = w1raw + jnp.where(top1[e], sc[e], zero)
        w2raw = w2raw + jnp.where(top2[e], sc[e], zero)
    inv = SCALE / (w1raw + w2raw + 1e-20)
    combT = jnp.concatenate(
        [(jnp.where(top1[e], w1raw, zero)
          + jnp.where(top2[e], w2raw, zero)) * inv
         for e in range(E)], axis=0)
    return combT.T


def _fused_kernel(x_ref, wgate_ref, bias_ref, wg_hbm, wu_hbm, wd_hbm,
                  wsg_hbm, wsu_hbm, wsd_hbm, out_ref,
                  g_stg, u_stg, d_stg, xb_ref, comb_ref,
                  act_ref, wd2_ref, sems):
    def copies(e, b):
        if e < E:
            return (pltpu.make_async_copy(wg_hbm.at[e], g_stg.at[b], sems.at[b]),
                    pltpu.make_async_copy(wu_hbm.at[e], u_stg.at[b], sems.at[b]),
                    pltpu.make_async_copy(wd_hbm.at[e], d_stg.at[b], sems.at[b]))
        return (pltpu.make_async_copy(wsg_hbm, g_stg.at[b], sems.at[b]),
                pltpu.make_async_copy(wsu_hbm, u_stg.at[b], sems.at[b]),
                pltpu.make_async_copy(wsd_hbm, d_stg.at[b], sems.at[b]))

    for c in copies(0, 0):
        c.start()

    x = x_ref[...]
    comb_ref[...] = _routing_cols(x, wgate_ref[...], bias_ref)
    xb_ref[...] = x.astype(jnp.bfloat16)
    for e in range(E + 1):
        b = e % 2
        if e < E:
            for c in copies(e + 1, 1 - b):
                c.start()
        for c in copies(e, b):
            c.wait()
        xb = xb_ref[...]
        wgb = g_stg[b].astype(jnp.bfloat16)
        wub = u_stg[b].astype(jnp.bfloat16)
        wd2_ref[:, e * DFF:(e + 1) * DFF] = d_stg[b].astype(jnp.bfloat16)
        gm = lax.dot_general(xb, wgb, _RHS_T,
                             preferred_element_type=jnp.float32)
        um = lax.dot_general(xb, wub, _RHS_T,
                             preferred_element_type=jnp.float32)
        act = gm * jax.nn.sigmoid(gm) * um
        if e < E:
            act = act * comb_ref[:, e:e + 1]
        act_ref[:, e * DFF:(e + 1) * DFF] = act.astype(jnp.bfloat16)
    out_ref[...] = lax.dot_general(act_ref[...], wd2_ref[...], _RHS_T,
                                   preferred_element_type=jnp.float32)


def kernel(hidden_states, image_mask, audio_mask, Wg, expert_bias,
           W_gate, W_up, W_down, Ws_gate, Ws_up, Ws_down):
    orig_shape = hidden_states.shape
    x = hidden_states.reshape(-1, H)
    bias2 = expert_bias.reshape(1, E)

    out = pl.pallas_call(
        _fused_kernel,
        in_specs=[
            pl.BlockSpec((N, H), lambda: (0, 0)),
            pl.BlockSpec((E, H), lambda: (0, 0)),
            pl.BlockSpec((1, E), lambda: (0, 0)),
            pl.BlockSpec(memory_space=pl.ANY),
            pl.BlockSpec(memory_space=pl.ANY),
            pl.BlockSpec(memory_space=pl.ANY),
            pl.BlockSpec(memory_space=pl.ANY),
            pl.BlockSpec(memory_space=pl.ANY),
            pl.BlockSpec(memory_space=pl.ANY),
        ],
        out_specs=pl.BlockSpec((N, H), lambda: (0, 0)),
        out_shape=jax.ShapeDtypeStruct((N, H), jnp.float32),
        scratch_shapes=[
            pltpu.VMEM((2, DFF, H), jnp.float32),
            pltpu.VMEM((2, DFF, H), jnp.float32),
            pltpu.VMEM((2, H, DFF), jnp.float32),
            pltpu.VMEM((N, H), jnp.bfloat16),
            pltpu.VMEM((N, E), jnp.float32),
            pltpu.VMEM((N, (E + 1) * DFF), jnp.bfloat16),
            pltpu.VMEM((H, (E + 1) * DFF), jnp.bfloat16),
            pltpu.SemaphoreType.DMA((2,)),
        ],
    )(x, Wg, bias2, W_gate, W_up, W_down, Ws_gate, Ws_up, Ws_down)

    return out.reshape(orig_shape)
